# baseline (speedup 1.0000x reference)
import functools
import jax
import jax.numpy as jnp
from jax import lax
from jax.experimental import pallas as pl
from jax.experimental.pallas import tpu as pltpu
from jax.experimental.pallas import tpu_sc as plsc

N, E, D, R, B = 10000, 320000, 128, 237, 1024

NC, NS = 2, 16
NW = NC * NS
NPAD = 10240
ROWS_PT = NPAD // NS
EW = 64
EPT = -(-(-(-E // (EW * NW))) // 8) * 8
E_ROWS = EPT * NW
E_PAD = E_ROWS * EW
BPT = B // NW

_mesh = plsc.VectorSubcoreMesh(
    core_axis_name="c", subcore_axis_name="s", num_cores=NC, num_subcores=NS)


@functools.partial(
    pl.kernel,
    out_type=[jax.ShapeDtypeStruct((NC, NPAD, 128), jnp.float32),
              jax.ShapeDtypeStruct((NC, NPAD, 128), jnp.float32)],
    mesh=_mesh,
    scratch_types=[
        pltpu.VMEM((EPT, EW), jnp.int32),
        pltpu.VMEM((EPT, EW), jnp.int32),
        pltpu.VMEM((EW, 128), jnp.float32),
        pltpu.VMEM_SHARED((NPAD, 128), jnp.float32),
        pltpu.SemaphoreType.DMA,
    ],
)
def _sc_degrees(src_hbm, dst_hbm, ones_hbm, zeros_hbm, dout_hbm, din_hbm,
                src_v, dst_v, ones_v, cnt_s, sem):
    c = lax.axis_index("c")
    s = lax.axis_index("s")
    wid = c * NS + s
    pltpu.sync_copy(src_hbm.at[pl.ds(wid * EPT, EPT)], src_v)
    pltpu.sync_copy(dst_hbm.at[pl.ds(wid * EPT, EPT)], dst_v)
    pltpu.sync_copy(ones_hbm, ones_v)
    pltpu.sync_copy(zeros_hbm.at[pl.ds(s * ROWS_PT, ROWS_PT)],
                    cnt_s.at[pl.ds(s * ROWS_PT, ROWS_PT)])
    plsc.subcore_barrier()

    def body_src(j, carry):
        pltpu.sync_copy(ones_v, cnt_s.at[src_v.at[j]], add=True)
        return carry

    lax.fori_loop(0, EPT, body_src, 0)
    plsc.subcore_barrier()
    pltpu.sync_copy(cnt_s.at[pl.ds(s * ROWS_PT, ROWS_PT)],
                    dout_hbm.at[c, pl.ds(s * ROWS_PT, ROWS_PT)])
    pltpu.sync_copy(zeros_hbm.at[pl.ds(s * ROWS_PT, ROWS_PT)],
                    cnt_s.at[pl.ds(s * ROWS_PT, ROWS_PT)])
    plsc.subcore_barrier()

    def body_dst(j, carry):
        pltpu.sync_copy(ones_v, cnt_s.at[dst_v.at[j]], add=True)
        return carry

    lax.fori_loop(0, EPT, body_dst, 0)
    plsc.subcore_barrier()
    pltpu.sync_copy(cnt_s.at[pl.ds(s * ROWS_PT, ROWS_PT)],
                    din_hbm.at[c, pl.ds(s * ROWS_PT, ROWS_PT)])


NP2 = 10
NSTR = 4


@functools.partial(
    pl.kernel,
    out_type=jax.ShapeDtypeStruct((NC, NPAD, D), jnp.float32),
    mesh=_mesh,
    scratch_types=[
        pltpu.VMEM((EPT // NP2, EW), jnp.int32),
        pltpu.VMEM((EPT // NP2, EW), jnp.int32),
    ] + [pltpu.VMEM((EW, D), jnp.float32)] * NSTR + [
        pltpu.VMEM_SHARED((NPAD, D), jnp.float32),
    ] + [pltpu.SemaphoreType.DMA] * (2 * NSTR),
)
def _sc_edge_agg(m_hbm, src_hbm, dst_hbm, zeros_hbm, agg_hbm,
                 src_v, dst_v, *rest):
    rows = rest[:NSTR]
    agg_s = rest[NSTR]
    gsem = rest[NSTR + 1:NSTR + 1 + NSTR]
    ssem = rest[NSTR + 1 + NSTR:]
    c = lax.axis_index("c")
    s = lax.axis_index("s")
    wid = c * NS + s
    per = EPT // NP2
    pltpu.sync_copy(zeros_hbm.at[pl.ds(s * ROWS_PT, ROWS_PT)],
                    agg_s.at[pl.ds(s * ROWS_PT, ROWS_PT)])
    plsc.subcore_barrier()

    for phase in range(NP2):
        pltpu.sync_copy(src_hbm.at[pl.ds(wid * EPT + phase * per, per)],
                        src_v)
        pltpu.sync_copy(dst_hbm.at[pl.ds(wid * EPT + phase * per, per)],
                        dst_v)

        def body(i, carry):
            jj = i * NSTR
            hs = [pltpu.async_copy(m_hbm.at[src_v.at[jj + k]], rows[k],
                                   gsem[k]) for k in range(NSTR)]
            ss = []
            for k in range(NSTR):
                hs[k].wait()
                ss.append(pltpu.async_copy(
                    rows[k], agg_s.at[dst_v.at[jj + k]], ssem[k], add=True))
            for k in range(NSTR):
                ss[k].wait()
            return carry

        lax.fori_loop(0, per // NSTR, body, 0)

    plsc.subcore_barrier()
    pltpu.sync_copy(agg_s.at[pl.ds(s * ROWS_PT, ROWS_PT)],
                    agg_hbm.at[c, pl.ds(s * ROWS_PT, ROWS_PT)])


@functools.partial(
    pl.kernel,
    out_type=[jax.ShapeDtypeStruct((B, D), jnp.float32),
              jax.ShapeDtypeStruct((B, D), jnp.float32)],
    mesh=_mesh,
    scratch_types=[
        pltpu.VMEM((NW, BPT), jnp.int32),
        pltpu.VMEM((NW, BPT), jnp.int32),
        pltpu.VMEM((BPT, D), jnp.float32),
        pltpu.VMEM((BPT, D), jnp.float32),
        pltpu.SemaphoreType.DMA,
    ],
)
def _sc_batch_gather(h_hbm, rt_hbm, e1_hbm, rel_hbm, eout_hbm, rout_hbm,
                     e1_v, rel_v, erows_v, rrows_v, sem):
    c = lax.axis_index("c")
    s = lax.axis_index("s")
    wid = c * NS + s
    pltpu.sync_copy(e1_hbm, e1_v)
    pltpu.sync_copy(rel_hbm, rel_v)
    pltpu.async_copy(h_hbm.at[e1_v.at[wid]], erows_v, sem).wait()
    pltpu.async_copy(rt_hbm.at[rel_v.at[wid]], rrows_v, sem).wait()
    pltpu.sync_copy(erows_v, eout_hbm.at[pl.ds(wid * BPT, BPT)])
    pltpu.sync_copy(rrows_v, rout_hbm.at[pl.ds(wid * BPT, BPT)])


def _tc_deg_prep_body(dop_ref, dip_ref, cout_ref, cin_ref):
    deg_o = dop_ref[0] + dop_ref[1]
    deg_i = dip_ref[0] + dip_ref[1]
    cout_ref[...] = lax.rsqrt(jnp.maximum(deg_o, 1.0))
    cin_ref[...] = lax.rsqrt(jnp.maximum(deg_i, 1.0))


def _tc_deg_prep(degp_out, degp_in):
    grid = NPAD // 128
    return pl.pallas_call(
        _tc_deg_prep_body,
        grid=(grid,),
        in_specs=[
            pl.BlockSpec((NC, 128, 128), lambda i: (0, i, 0)),
            pl.BlockSpec((NC, 128, 128), lambda i: (0, i, 0)),
        ],
        out_specs=[
            pl.BlockSpec((128, 128), lambda i: (i, 0)),
            pl.BlockSpec((128, 128), lambda i: (i, 0)),
        ],
        out_shape=[jax.ShapeDtypeStruct((NPAD, 128), jnp.float32),
                   jax.ShapeDtypeStruct((NPAD, 128), jnp.float32)],
    )(degp_out, degp_in)


def _tc_prescale_body(nf_ref, w_ref, cout_ref, out_ref):
    hw = jnp.dot(nf_ref[...], w_ref[...], preferred_element_type=jnp.float32)
    out_ref[...] = hw * cout_ref[...]


def _tc_prescale(h, w, coutb):
    grid = NPAD // 128
    return pl.pallas_call(
        _tc_prescale_body,
        grid=(grid,),
        in_specs=[
            pl.BlockSpec((128, D), lambda i: (i, 0)),
            pl.BlockSpec((D, D), lambda i: (0, 0)),
            pl.BlockSpec((128, D), lambda i: (i, 0)),
        ],
        out_specs=pl.BlockSpec((128, D), lambda i: (i, 0)),
        out_shape=jax.ShapeDtypeStruct((NPAD, D), jnp.float32),
    )(h, w, coutb)


def _tc_postscale_body(aggp_ref, cin_ref, b_ref, x_ref, stats_ref):
    i = pl.program_id(0)
    agg = aggp_ref[0] + aggp_ref[1]
    x = agg * cin_ref[...] + b_ref[0][None, :]
    x_ref[...] = x
    rows = lax.broadcasted_iota(jnp.int32, (128, 1), 0) + i * 128
    xm = jnp.where(rows < N, x, 0.0)
    s1 = jnp.sum(xm, axis=0)
    s2 = jnp.sum(xm * xm, axis=0)

    @pl.when(i == 0)
    def _():
        stats_ref[...] = jnp.zeros((8, D), jnp.float32)

    stats_ref[0, :] += s1
    stats_ref[1, :] += s2


def _tc_postscale(aggp, cinb, b):
    grid = NPAD // 128
    return pl.pallas_call(
        _tc_postscale_body,
        grid=(grid,),
        in_specs=[
            pl.BlockSpec((NC, 128, D), lambda i: (0, i, 0)),
            pl.BlockSpec((128, D), lambda i: (i, 0)),
            pl.BlockSpec((1, D), lambda i: (0, 0)),
        ],
        out_specs=[
            pl.BlockSpec((128, D), lambda i: (i, 0)),
            pl.BlockSpec((8, D), lambda i: (0, 0)),
        ],
        out_shape=[jax.ShapeDtypeStruct((NPAD, D), jnp.float32),
                   jax.ShapeDtypeStruct((8, D), jnp.float32)],
    )(aggp, cinb, b)


def _tc_bn_matmul_body(x_ref, stats_ref, g_ref, be_ref, w_ref, cout_ref,
                       out_ref):
    mu = stats_ref[0, :] * (1.0 / N)
    var = stats_ref[1, :] * (1.0 / N) - mu * mu
    scale = g_ref[0] * lax.rsqrt(var + 1e-5)
    h = jnp.tanh((x_ref[...] - mu[None, :]) * scale[None, :] + be_ref[0][None, :])
    hw = jnp.dot(h, w_ref[...], preferred_element_type=jnp.float32)
    out_ref[...] = hw * cout_ref[...]


def _tc_bn_matmul(x, stats, g, be, w, coutb):
    grid = NPAD // 128
    return pl.pallas_call(
        _tc_bn_matmul_body,
        grid=(grid,),
        in_specs=[
            pl.BlockSpec((128, D), lambda i: (i, 0)),
            pl.BlockSpec((8, D), lambda i: (0, 0)),
            pl.BlockSpec((1, D), lambda i: (0, 0)),
            pl.BlockSpec((1, D), lambda i: (0, 0)),
            pl.BlockSpec((D, D), lambda i: (0, 0)),
            pl.BlockSpec((128, D), lambda i: (i, 0)),
        ],
        out_specs=pl.BlockSpec((128, D), lambda i: (i, 0)),
        out_shape=jax.ShapeDtypeStruct((NPAD, D), jnp.float32),
    )(x, stats, g, be, w, coutb)


def _tc_bn_body(x_ref, stats_ref, g_ref, be_ref, out_ref):
    mu = stats_ref[0, :] * (1.0 / N)
    var = stats_ref[1, :] * (1.0 / N) - mu * mu
    scale = g_ref[0] * lax.rsqrt(var + 1e-5)
    out_ref[...] = jnp.tanh(
        (x_ref[...] - mu[None, :]) * scale[None, :] + be_ref[0][None, :])


def _tc_bn(x, stats, g, be):
    grid = NPAD // 128
    return pl.pallas_call(
        _tc_bn_body,
        grid=(grid,),
        in_specs=[
            pl.BlockSpec((128, D), lambda i: (i, 0)),
            pl.BlockSpec((8, D), lambda i: (0, 0)),
            pl.BlockSpec((1, D), lambda i: (0, 0)),
            pl.BlockSpec((1, D), lambda i: (0, 0)),
        ],
        out_specs=pl.BlockSpec((128, D), lambda i: (i, 0)),
        out_shape=jax.ShapeDtypeStruct((NPAD, D), jnp.float32),
    )(x, stats, g, be)


def _tc_score_body(e_ref, r_ref, h_ref, out_ref):
    q = e_ref[...] * r_ref[...]
    logits = lax.dot_general(q, h_ref[...], (((1,), (1,)), ((), ())),
                             preferred_element_type=jnp.float32)
    out_ref[...] = 1.0 / (1.0 + jnp.exp(-logits))


def _tc_score(e1_emb, r_emb, h2):
    grid = NPAD // 128
    return pl.pallas_call(
        _tc_score_body,
        grid=(grid,),
        in_specs=[
            pl.BlockSpec((B, D), lambda j: (0, 0)),
            pl.BlockSpec((B, D), lambda j: (0, 0)),
            pl.BlockSpec((128, D), lambda j: (j, 0)),
        ],
        out_specs=pl.BlockSpec((B, 128), lambda j: (0, j)),
        out_shape=jax.ShapeDtypeStruct((B, NPAD), jnp.float32),
    )(e1_emb, r_emb, h2)


@jax.jit
def kernel(node_feat, edge_index, e1, rel, W0, b0, W1, b1,
           g0, be0, g1, be1, rel_table):
    src = edge_index[0].astype(jnp.int32)
    dst = edge_index[1].astype(jnp.int32)
    pad = E_PAD - E
    src_p = jnp.concatenate([src, jnp.full((pad,), N, jnp.int32)]).reshape(E_ROWS, EW)
    dst_p = jnp.concatenate([dst, jnp.full((pad,), N, jnp.int32)]).reshape(E_ROWS, EW)

    nf_p = jnp.concatenate(
        [node_feat, jnp.zeros((NPAD - N, D), jnp.float32)], axis=0)
    zerosD = jnp.zeros((NPAD, D), jnp.float32)
    onesD = jnp.ones((EW, 128), jnp.float32)
    e1_2d = e1.astype(jnp.int32).reshape(NW, BPT)
    rel_2d = rel.astype(jnp.int32).reshape(NW, BPT)
    b0_2d = b0.reshape(1, D)
    b1_2d = b1.reshape(1, D)
    g0_2d = g0.reshape(1, D)
    g1_2d = g1.reshape(1, D)
    be0_2d = be0.reshape(1, D)
    be1_2d = be1.reshape(1, D)

    degp_out, degp_in = _sc_degrees(src_p, dst_p, onesD, zerosD)
    coutb, cinb = _tc_deg_prep(degp_out, degp_in)

    m0 = _tc_prescale(nf_p, W0, coutb)
    aggp0 = _sc_edge_agg(m0, src_p, dst_p, zerosD)
    x0, stats0 = _tc_postscale(aggp0, cinb, b0_2d)
    m1 = _tc_bn_matmul(x0, stats0, g0_2d, be0_2d, W1, coutb)
    aggp1 = _sc_edge_agg(m1, src_p, dst_p, zerosD)
    x1, stats1 = _tc_postscale(aggp1, cinb, b1_2d)
    h2 = _tc_bn(x1, stats1, g1_2d, be1_2d)

    e1_emb, r_emb = _sc_batch_gather(h2, rel_table, e1_2d, rel_2d)
    pred = _tc_score(e1_emb, r_emb, h2)
    return pred[:, :N]


# speedup vs baseline: 1.1045x; 1.1045x over previous
import functools
import jax
import jax.numpy as jnp
from jax import lax
from jax.experimental import pallas as pl
from jax.experimental.pallas import tpu as pltpu
from jax.experimental.pallas import tpu_sc as plsc

N, E, D, R, B = 10000, 320000, 128, 237, 1024

NC, NS = 2, 16
NW = NC * NS
NPAD = 10240
ROWS_PT = NPAD // NS
EW = 128
N_PHASE = 5
EPT = -(-(-(-E // (EW * NW))) // 8) * 8
E_ROWS = EPT * NW
E_PAD = E_ROWS * EW
BPT = B // NW

_mesh = plsc.VectorSubcoreMesh(
    core_axis_name="c", subcore_axis_name="s", num_cores=NC, num_subcores=NS)


@functools.partial(
    pl.kernel,
    out_type=[jax.ShapeDtypeStruct((NC, NPAD, 128), jnp.float32),
              jax.ShapeDtypeStruct((NC, NPAD, 128), jnp.float32)],
    mesh=_mesh,
    scratch_types=[
        pltpu.VMEM((EPT, EW), jnp.int32),
        pltpu.VMEM((EPT, EW), jnp.int32),
        pltpu.VMEM((EW, 128), jnp.float32),
        pltpu.VMEM_SHARED((NPAD, 128), jnp.float32),
        pltpu.SemaphoreType.DMA,
    ],
)
def _sc_degrees(src_hbm, dst_hbm, ones_hbm, zeros_hbm, dout_hbm, din_hbm,
                src_v, dst_v, ones_v, cnt_s, sem):
    c = lax.axis_index("c")
    s = lax.axis_index("s")
    wid = c * NS + s
    pltpu.sync_copy(src_hbm.at[pl.ds(wid * EPT, EPT)], src_v)
    pltpu.sync_copy(dst_hbm.at[pl.ds(wid * EPT, EPT)], dst_v)
    pltpu.sync_copy(ones_hbm, ones_v)
    pltpu.sync_copy(zeros_hbm.at[pl.ds(s * ROWS_PT, ROWS_PT)],
                    cnt_s.at[pl.ds(s * ROWS_PT, ROWS_PT)])
    plsc.subcore_barrier()

    def body_src(j, carry):
        pltpu.sync_copy(ones_v, cnt_s.at[src_v.at[j]], add=True)
        return carry

    lax.fori_loop(0, EPT, body_src, 0)
    plsc.subcore_barrier()
    pltpu.sync_copy(cnt_s.at[pl.ds(s * ROWS_PT, ROWS_PT)],
                    dout_hbm.at[c, pl.ds(s * ROWS_PT, ROWS_PT)])
    pltpu.sync_copy(zeros_hbm.at[pl.ds(s * ROWS_PT, ROWS_PT)],
                    cnt_s.at[pl.ds(s * ROWS_PT, ROWS_PT)])
    plsc.subcore_barrier()

    def body_dst(j, carry):
        pltpu.sync_copy(ones_v, cnt_s.at[dst_v.at[j]], add=True)
        return carry

    lax.fori_loop(0, EPT, body_dst, 0)
    plsc.subcore_barrier()
    pltpu.sync_copy(cnt_s.at[pl.ds(s * ROWS_PT, ROWS_PT)],
                    din_hbm.at[c, pl.ds(s * ROWS_PT, ROWS_PT)])


@functools.partial(
    pl.kernel,
    out_type=jax.ShapeDtypeStruct((NC, NPAD, D), jnp.float32),
    mesh=_mesh,
    scratch_types=[
        pltpu.VMEM((EPT // N_PHASE, EW), jnp.int32),
        pltpu.VMEM((EPT // N_PHASE, EW), jnp.int32),
        pltpu.VMEM((EW, D), jnp.float32),
        pltpu.VMEM((EW, D), jnp.float32),
        pltpu.VMEM_SHARED((NPAD, D), jnp.float32),
        pltpu.SemaphoreType.DMA,
        pltpu.SemaphoreType.DMA,
        pltpu.SemaphoreType.DMA,
        pltpu.SemaphoreType.DMA,
    ],
)
def _sc_edge_agg(m_hbm, src_hbm, dst_hbm, zeros_hbm, agg_hbm,
                 src_v, dst_v, rows0_v, rows1_v, agg_s, sem0, sem1,
                 ssem0, ssem1):
    c = lax.axis_index("c")
    s = lax.axis_index("s")
    wid = c * NS + s
    per = EPT // N_PHASE
    pltpu.sync_copy(zeros_hbm.at[pl.ds(s * ROWS_PT, ROWS_PT)],
                    agg_s.at[pl.ds(s * ROWS_PT, ROWS_PT)])
    plsc.subcore_barrier()

    for phase in range(N_PHASE):
        pltpu.sync_copy(src_hbm.at[pl.ds(wid * EPT + phase * per, per)],
                        src_v)
        pltpu.sync_copy(dst_hbm.at[pl.ds(wid * EPT + phase * per, per)],
                        dst_v)

        def body(i, carry):
            jj = i * 2
            h0 = pltpu.async_copy(m_hbm.at[src_v.at[jj]], rows0_v, sem0)
            h1 = pltpu.async_copy(m_hbm.at[src_v.at[jj + 1]], rows1_v, sem1)
            h0.wait()
            s0 = pltpu.async_copy(rows0_v, agg_s.at[dst_v.at[jj]], ssem0,
                                  add=True)
            h1.wait()
            s1 = pltpu.async_copy(rows1_v, agg_s.at[dst_v.at[jj + 1]], ssem1,
                                  add=True)
            s0.wait()
            s1.wait()
            return carry

        lax.fori_loop(0, per // 2, body, 0)
    plsc.subcore_barrier()
    pltpu.sync_copy(agg_s.at[pl.ds(s * ROWS_PT, ROWS_PT)],
                    agg_hbm.at[c, pl.ds(s * ROWS_PT, ROWS_PT)])


@functools.partial(
    pl.kernel,
    out_type=[jax.ShapeDtypeStruct((B, D), jnp.float32),
              jax.ShapeDtypeStruct((B, D), jnp.float32)],
    mesh=_mesh,
    scratch_types=[
        pltpu.VMEM((NW, BPT), jnp.int32),
        pltpu.VMEM((NW, BPT), jnp.int32),
        pltpu.VMEM((BPT, D), jnp.float32),
        pltpu.VMEM((BPT, D), jnp.float32),
        pltpu.SemaphoreType.DMA,
    ],
)
def _sc_batch_gather(h_hbm, rt_hbm, e1_hbm, rel_hbm, eout_hbm, rout_hbm,
                     e1_v, rel_v, erows_v, rrows_v, sem):
    c = lax.axis_index("c")
    s = lax.axis_index("s")
    wid = c * NS + s
    pltpu.sync_copy(e1_hbm, e1_v)
    pltpu.sync_copy(rel_hbm, rel_v)
    pltpu.async_copy(h_hbm.at[e1_v.at[wid]], erows_v, sem).wait()
    pltpu.async_copy(rt_hbm.at[rel_v.at[wid]], rrows_v, sem).wait()
    pltpu.sync_copy(erows_v, eout_hbm.at[pl.ds(wid * BPT, BPT)])
    pltpu.sync_copy(rrows_v, rout_hbm.at[pl.ds(wid * BPT, BPT)])


def _tc_deg_prep_body(dop_ref, dip_ref, cout_ref, cin_ref):
    deg_o = dop_ref[0] + dop_ref[1]
    deg_i = dip_ref[0] + dip_ref[1]
    cout_ref[...] = lax.rsqrt(jnp.maximum(deg_o, 1.0))
    cin_ref[...] = lax.rsqrt(jnp.maximum(deg_i, 1.0))


def _tc_deg_prep(degp_out, degp_in):
    grid = NPAD // 128
    return pl.pallas_call(
        _tc_deg_prep_body,
        grid=(grid,),
        in_specs=[
            pl.BlockSpec((NC, 128, 128), lambda i: (0, i, 0)),
            pl.BlockSpec((NC, 128, 128), lambda i: (0, i, 0)),
        ],
        out_specs=[
            pl.BlockSpec((128, 128), lambda i: (i, 0)),
            pl.BlockSpec((128, 128), lambda i: (i, 0)),
        ],
        out_shape=[jax.ShapeDtypeStruct((NPAD, 128), jnp.float32),
                   jax.ShapeDtypeStruct((NPAD, 128), jnp.float32)],
    )(degp_out, degp_in)


def _tc_prescale_body(nf_ref, w_ref, cout_ref, out_ref):
    hw = jnp.dot(nf_ref[...], w_ref[...], preferred_element_type=jnp.float32)
    out_ref[...] = hw * cout_ref[...]


def _tc_prescale(h, w, coutb):
    grid = NPAD // 128
    return pl.pallas_call(
        _tc_prescale_body,
        grid=(grid,),
        in_specs=[
            pl.BlockSpec((128, D), lambda i: (i, 0)),
            pl.BlockSpec((D, D), lambda i: (0, 0)),
            pl.BlockSpec((128, D), lambda i: (i, 0)),
        ],
        out_specs=pl.BlockSpec((128, D), lambda i: (i, 0)),
        out_shape=jax.ShapeDtypeStruct((NPAD, D), jnp.float32),
    )(h, w, coutb)


def _tc_postscale_body(aggp_ref, cin_ref, b_ref, x_ref, stats_ref):
    i = pl.program_id(0)
    agg = aggp_ref[0] + aggp_ref[1]
    x = agg * cin_ref[...] + b_ref[0][None, :]
    x_ref[...] = x
    rows = lax.broadcasted_iota(jnp.int32, (128, 1), 0) + i * 128
    xm = jnp.where(rows < N, x, 0.0)
    s1 = jnp.sum(xm, axis=0)
    s2 = jnp.sum(xm * xm, axis=0)

    @pl.when(i == 0)
    def _():
        stats_ref[...] = jnp.zeros((8, D), jnp.float32)

    stats_ref[0, :] += s1
    stats_ref[1, :] += s2


def _tc_postscale(aggp, cinb, b):
    grid = NPAD // 128
    return pl.pallas_call(
        _tc_postscale_body,
        grid=(grid,),
        in_specs=[
            pl.BlockSpec((NC, 128, D), lambda i: (0, i, 0)),
            pl.BlockSpec((128, D), lambda i: (i, 0)),
            pl.BlockSpec((1, D), lambda i: (0, 0)),
        ],
        out_specs=[
            pl.BlockSpec((128, D), lambda i: (i, 0)),
            pl.BlockSpec((8, D), lambda i: (0, 0)),
        ],
        out_shape=[jax.ShapeDtypeStruct((NPAD, D), jnp.float32),
                   jax.ShapeDtypeStruct((8, D), jnp.float32)],
    )(aggp, cinb, b)


def _tc_bn_matmul_body(x_ref, stats_ref, g_ref, be_ref, w_ref, cout_ref, out_ref):
    mu = stats_ref[0, :] * (1.0 / N)
    var = stats_ref[1, :] * (1.0 / N) - mu * mu
    scale = g_ref[0] * lax.rsqrt(var + 1e-5)
    h = jnp.tanh((x_ref[...] - mu[None, :]) * scale[None, :] + be_ref[0][None, :])
    hw = jnp.dot(h, w_ref[...], preferred_element_type=jnp.float32)
    out_ref[...] = hw * cout_ref[...]


def _tc_bn_matmul(x, stats, g, be, w, coutb):
    grid = NPAD // 128
    return pl.pallas_call(
        _tc_bn_matmul_body,
        grid=(grid,),
        in_specs=[
            pl.BlockSpec((128, D), lambda i: (i, 0)),
            pl.BlockSpec((8, D), lambda i: (0, 0)),
            pl.BlockSpec((1, D), lambda i: (0, 0)),
            pl.BlockSpec((1, D), lambda i: (0, 0)),
            pl.BlockSpec((D, D), lambda i: (0, 0)),
            pl.BlockSpec((128, D), lambda i: (i, 0)),
        ],
        out_specs=pl.BlockSpec((128, D), lambda i: (i, 0)),
        out_shape=jax.ShapeDtypeStruct((NPAD, D), jnp.float32),
    )(x, stats, g, be, w, coutb)


def _tc_bn_body(x_ref, stats_ref, g_ref, be_ref, out_ref):
    mu = stats_ref[0, :] * (1.0 / N)
    var = stats_ref[1, :] * (1.0 / N) - mu * mu
    scale = g_ref[0] * lax.rsqrt(var + 1e-5)
    out_ref[...] = jnp.tanh(
        (x_ref[...] - mu[None, :]) * scale[None, :] + be_ref[0][None, :])


def _tc_bn(x, stats, g, be):
    grid = NPAD // 128
    return pl.pallas_call(
        _tc_bn_body,
        grid=(grid,),
        in_specs=[
            pl.BlockSpec((128, D), lambda i: (i, 0)),
            pl.BlockSpec((8, D), lambda i: (0, 0)),
            pl.BlockSpec((1, D), lambda i: (0, 0)),
            pl.BlockSpec((1, D), lambda i: (0, 0)),
        ],
        out_specs=pl.BlockSpec((128, D), lambda i: (i, 0)),
        out_shape=jax.ShapeDtypeStruct((NPAD, D), jnp.float32),
    )(x, stats, g, be)


def _tc_score_body(e_ref, r_ref, h_ref, out_ref):
    q = e_ref[...] * r_ref[...]
    logits = lax.dot_general(q, h_ref[...], (((1,), (1,)), ((), ())),
                             preferred_element_type=jnp.float32)
    out_ref[...] = 1.0 / (1.0 + jnp.exp(-logits))


def _tc_score(e1_emb, r_emb, h2):
    grid = NPAD // 128
    return pl.pallas_call(
        _tc_score_body,
        grid=(grid,),
        in_specs=[
            pl.BlockSpec((B, D), lambda j: (0, 0)),
            pl.BlockSpec((B, D), lambda j: (0, 0)),
            pl.BlockSpec((128, D), lambda j: (j, 0)),
        ],
        out_specs=pl.BlockSpec((B, 128), lambda j: (0, j)),
        out_shape=jax.ShapeDtypeStruct((B, NPAD), jnp.float32),
    )(e1_emb, r_emb, h2)


@jax.jit
def kernel(node_feat, edge_index, e1, rel, W0, b0, W1, b1,
           g0, be0, g1, be1, rel_table):
    src = edge_index[0].astype(jnp.int32)
    dst = edge_index[1].astype(jnp.int32)
    pad = E_PAD - E
    src_p = jnp.concatenate([src, jnp.full((pad,), N, jnp.int32)]).reshape(E_ROWS, EW)
    dst_p = jnp.concatenate([dst, jnp.full((pad,), N, jnp.int32)]).reshape(E_ROWS, EW)

    nf_p = jnp.concatenate(
        [node_feat, jnp.zeros((NPAD - N, D), jnp.float32)], axis=0)
    zerosD = jnp.zeros((NPAD, D), jnp.float32)
    onesD = jnp.ones((EW, 128), jnp.float32)
    e1_2d = e1.astype(jnp.int32).reshape(NW, BPT)
    rel_2d = rel.astype(jnp.int32).reshape(NW, BPT)
    b0_2d = b0.reshape(1, D)
    b1_2d = b1.reshape(1, D)
    g0_2d = g0.reshape(1, D)
    g1_2d = g1.reshape(1, D)
    be0_2d = be0.reshape(1, D)
    be1_2d = be1.reshape(1, D)

    degp_out, degp_in = _sc_degrees(src_p, dst_p, onesD, zerosD)
    coutb, cinb = _tc_deg_prep(degp_out, degp_in)

    m0 = _tc_prescale(nf_p, W0, coutb)
    aggp0 = _sc_edge_agg(m0, src_p, dst_p, zerosD)
    x0, stats0 = _tc_postscale(aggp0, cinb, b0_2d)
    m1 = _tc_bn_matmul(x0, stats0, g0_2d, be0_2d, W1, coutb)
    aggp1 = _sc_edge_agg(m1, src_p, dst_p, zerosD)
    x1, stats1 = _tc_postscale(aggp1, cinb, b1_2d)
    h2 = _tc_bn(x1, stats1, g1_2d, be1_2d)

    e1_emb, r_emb = _sc_batch_gather(h2, rel_table, e1_2d, rel_2d)
    pred = _tc_score(e1_emb, r_emb, h2)
    return pred[:, :N]


# speedup vs baseline: 1.2079x; 1.0936x over previous
import functools
import jax
import jax.numpy as jnp
from jax import lax
from jax.experimental import pallas as pl
from jax.experimental.pallas import tpu as pltpu
from jax.experimental.pallas import tpu_sc as plsc

N, E, D, R, B = 10000, 320000, 128, 237, 1024

NC, NS = 2, 16
NW = NC * NS
NPAD = 10240
ROWS_PT = NPAD // NS
EW = 128
N_PHASE = 5
EPT = -(-(-(-E // (EW * NW))) // 8) * 8
E_ROWS = EPT * NW
E_PAD = E_ROWS * EW
BPT = B // NW

_mesh = plsc.VectorSubcoreMesh(
    core_axis_name="c", subcore_axis_name="s", num_cores=NC, num_subcores=NS)


@functools.partial(
    pl.kernel,
    out_type=[jax.ShapeDtypeStruct((NC, NPAD, 128), jnp.float32),
              jax.ShapeDtypeStruct((NC, NPAD, 128), jnp.float32)],
    mesh=_mesh,
    scratch_types=[
        pltpu.VMEM((EPT, EW), jnp.int32),
        pltpu.VMEM((EPT, EW), jnp.int32),
        pltpu.VMEM((EW, 128), jnp.float32),
        pltpu.VMEM_SHARED((NPAD, 128), jnp.float32),
        pltpu.SemaphoreType.DMA,
    ],
)
def _sc_degrees(src_hbm, dst_hbm, ones_hbm, zeros_hbm, dout_hbm, din_hbm,
                src_v, dst_v, ones_v, cnt_s, sem):
    c = lax.axis_index("c")
    s = lax.axis_index("s")
    wid = c * NS + s
    pltpu.sync_copy(src_hbm.at[pl.ds(wid * EPT, EPT)], src_v)
    pltpu.sync_copy(dst_hbm.at[pl.ds(wid * EPT, EPT)], dst_v)
    pltpu.sync_copy(ones_hbm, ones_v)
    pltpu.sync_copy(zeros_hbm.at[pl.ds(s * ROWS_PT, ROWS_PT)],
                    cnt_s.at[pl.ds(s * ROWS_PT, ROWS_PT)])
    plsc.subcore_barrier()

    def body_src(j, carry):
        pltpu.sync_copy(ones_v, cnt_s.at[src_v.at[j]], add=True)
        return carry

    lax.fori_loop(0, EPT, body_src, 0)
    plsc.subcore_barrier()
    pltpu.sync_copy(cnt_s.at[pl.ds(s * ROWS_PT, ROWS_PT)],
                    dout_hbm.at[c, pl.ds(s * ROWS_PT, ROWS_PT)])
    pltpu.sync_copy(zeros_hbm.at[pl.ds(s * ROWS_PT, ROWS_PT)],
                    cnt_s.at[pl.ds(s * ROWS_PT, ROWS_PT)])
    plsc.subcore_barrier()

    def body_dst(j, carry):
        pltpu.sync_copy(ones_v, cnt_s.at[dst_v.at[j]], add=True)
        return carry

    lax.fori_loop(0, EPT, body_dst, 0)
    plsc.subcore_barrier()
    pltpu.sync_copy(cnt_s.at[pl.ds(s * ROWS_PT, ROWS_PT)],
                    din_hbm.at[c, pl.ds(s * ROWS_PT, ROWS_PT)])


EPT_F = 120
EPT_S = 2 * EPT - EPT_F


@functools.partial(
    pl.kernel,
    out_type=jax.ShapeDtypeStruct((NC, NPAD, D), jnp.float32),
    mesh=_mesh,
    scratch_types=[
        pltpu.VMEM((EPT_F // N_PHASE, EW), jnp.int32),
        pltpu.VMEM((EPT_F // N_PHASE, EW), jnp.int32),
        pltpu.VMEM((EW, D), jnp.float32),
        pltpu.VMEM((EW, D), jnp.float32),
        pltpu.VMEM_SHARED((NPAD, D), jnp.float32),
        pltpu.SemaphoreType.DMA,
        pltpu.SemaphoreType.DMA,
        pltpu.SemaphoreType.DMA,
        pltpu.SemaphoreType.DMA,
    ],
)
def _sc_edge_agg(m_hbm, src_hbm, dst_hbm, zeros_hbm, agg_hbm,
                 src_v, dst_v, rows0_v, rows1_v, agg_s, sem0, sem1,
                 ssem0, ssem1):
    c = lax.axis_index("c")
    s = lax.axis_index("s")
    pltpu.sync_copy(zeros_hbm.at[pl.ds(s * ROWS_PT, ROWS_PT)],
                    agg_s.at[pl.ds(s * ROWS_PT, ROWS_PT)])
    plsc.subcore_barrier()

    def run(base, ept_c):
        per = ept_c // N_PHASE
        for phase in range(N_PHASE):
            pltpu.sync_copy(src_hbm.at[pl.ds(base + phase * per, per)],
                            src_v.at[pl.ds(0, per)])
            pltpu.sync_copy(dst_hbm.at[pl.ds(base + phase * per, per)],
                            dst_v.at[pl.ds(0, per)])

            def body(i, carry):
                jj = i * 2
                h0 = pltpu.async_copy(m_hbm.at[src_v.at[jj]], rows0_v, sem0)
                h1 = pltpu.async_copy(m_hbm.at[src_v.at[jj + 1]], rows1_v,
                                      sem1)
                h0.wait()
                s0 = pltpu.async_copy(rows0_v, agg_s.at[dst_v.at[jj]], ssem0,
                                      add=True)
                h1.wait()
                s1 = pltpu.async_copy(rows1_v, agg_s.at[dst_v.at[jj + 1]],
                                      ssem1, add=True)
                s0.wait()
                s1.wait()
                return carry

            lax.fori_loop(0, per // 2, body, 0)

    @pl.when(c == 0)
    def _():
        run(s * EPT_F, EPT_F)

    @pl.when(c == 1)
    def _():
        run(NS * EPT_F + s * EPT_S, EPT_S)

    plsc.subcore_barrier()
    pltpu.sync_copy(agg_s.at[pl.ds(s * ROWS_PT, ROWS_PT)],
                    agg_hbm.at[c, pl.ds(s * ROWS_PT, ROWS_PT)])


@functools.partial(
    pl.kernel,
    out_type=[jax.ShapeDtypeStruct((B, D), jnp.float32),
              jax.ShapeDtypeStruct((B, D), jnp.float32)],
    mesh=_mesh,
    scratch_types=[
        pltpu.VMEM((NW, BPT), jnp.int32),
        pltpu.VMEM((NW, BPT), jnp.int32),
        pltpu.VMEM((BPT, D), jnp.float32),
        pltpu.VMEM((BPT, D), jnp.float32),
        pltpu.SemaphoreType.DMA,
    ],
)
def _sc_batch_gather(h_hbm, rt_hbm, e1_hbm, rel_hbm, eout_hbm, rout_hbm,
                     e1_v, rel_v, erows_v, rrows_v, sem):
    c = lax.axis_index("c")
    s = lax.axis_index("s")
    wid = c * NS + s
    pltpu.sync_copy(e1_hbm, e1_v)
    pltpu.sync_copy(rel_hbm, rel_v)
    pltpu.async_copy(h_hbm.at[e1_v.at[wid]], erows_v, sem).wait()
    pltpu.async_copy(rt_hbm.at[rel_v.at[wid]], rrows_v, sem).wait()
    pltpu.sync_copy(erows_v, eout_hbm.at[pl.ds(wid * BPT, BPT)])
    pltpu.sync_copy(rrows_v, rout_hbm.at[pl.ds(wid * BPT, BPT)])


def _tc_deg_prep_body(dop_ref, dip_ref, cout_ref, cin_ref):
    deg_o = dop_ref[0] + dop_ref[1]
    deg_i = dip_ref[0] + dip_ref[1]
    cout_ref[...] = lax.rsqrt(jnp.maximum(deg_o, 1.0))
    cin_ref[...] = lax.rsqrt(jnp.maximum(deg_i, 1.0))


def _tc_deg_prep(degp_out, degp_in):
    grid = NPAD // 128
    return pl.pallas_call(
        _tc_deg_prep_body,
        grid=(grid,),
        in_specs=[
            pl.BlockSpec((NC, 128, 128), lambda i: (0, i, 0)),
            pl.BlockSpec((NC, 128, 128), lambda i: (0, i, 0)),
        ],
        out_specs=[
            pl.BlockSpec((128, 128), lambda i: (i, 0)),
            pl.BlockSpec((128, 128), lambda i: (i, 0)),
        ],
        out_shape=[jax.ShapeDtypeStruct((NPAD, 128), jnp.float32),
                   jax.ShapeDtypeStruct((NPAD, 128), jnp.float32)],
    )(degp_out, degp_in)


def _tc_prescale_body(nf_ref, w_ref, cout_ref, out_ref):
    hw = jnp.dot(nf_ref[...], w_ref[...], preferred_element_type=jnp.float32)
    out_ref[...] = hw * cout_ref[...]


def _tc_prescale(h, w, coutb):
    grid = NPAD // 128
    return pl.pallas_call(
        _tc_prescale_body,
        grid=(grid,),
        in_specs=[
            pl.BlockSpec((128, D), lambda i: (i, 0)),
            pl.BlockSpec((D, D), lambda i: (0, 0)),
            pl.BlockSpec((128, D), lambda i: (i, 0)),
        ],
        out_specs=pl.BlockSpec((128, D), lambda i: (i, 0)),
        out_shape=jax.ShapeDtypeStruct((NPAD, D), jnp.float32),
    )(h, w, coutb)


def _tc_postscale_body(aggp_ref, cin_ref, b_ref, x_ref, stats_ref):
    i = pl.program_id(0)
    agg = aggp_ref[0] + aggp_ref[1]
    x = agg * cin_ref[...] + b_ref[0][None, :]
    x_ref[...] = x
    rows = lax.broadcasted_iota(jnp.int32, (128, 1), 0) + i * 128
    xm = jnp.where(rows < N, x, 0.0)
    s1 = jnp.sum(xm, axis=0)
    s2 = jnp.sum(xm * xm, axis=0)

    @pl.when(i == 0)
    def _():
        stats_ref[...] = jnp.zeros((8, D), jnp.float32)

    stats_ref[0, :] += s1
    stats_ref[1, :] += s2


def _tc_postscale(aggp, cinb, b):
    grid = NPAD // 128
    return pl.pallas_call(
        _tc_postscale_body,
        grid=(grid,),
        in_specs=[
            pl.BlockSpec((NC, 128, D), lambda i: (0, i, 0)),
            pl.BlockSpec((128, D), lambda i: (i, 0)),
            pl.BlockSpec((1, D), lambda i: (0, 0)),
        ],
        out_specs=[
            pl.BlockSpec((128, D), lambda i: (i, 0)),
            pl.BlockSpec((8, D), lambda i: (0, 0)),
        ],
        out_shape=[jax.ShapeDtypeStruct((NPAD, D), jnp.float32),
                   jax.ShapeDtypeStruct((8, D), jnp.float32)],
    )(aggp, cinb, b)


def _tc_bn_matmul_body(x_ref, stats_ref, g_ref, be_ref, w_ref, cout_ref, out_ref):
    mu = stats_ref[0, :] * (1.0 / N)
    var = stats_ref[1, :] * (1.0 / N) - mu * mu
    scale = g_ref[0] * lax.rsqrt(var + 1e-5)
    h = jnp.tanh((x_ref[...] - mu[None, :]) * scale[None, :] + be_ref[0][None, :])
    hw = jnp.dot(h, w_ref[...], preferred_element_type=jnp.float32)
    out_ref[...] = hw * cout_ref[...]


def _tc_bn_matmul(x, stats, g, be, w, coutb):
    grid = NPAD // 128
    return pl.pallas_call(
        _tc_bn_matmul_body,
        grid=(grid,),
        in_specs=[
            pl.BlockSpec((128, D), lambda i: (i, 0)),
            pl.BlockSpec((8, D), lambda i: (0, 0)),
            pl.BlockSpec((1, D), lambda i: (0, 0)),
            pl.BlockSpec((1, D), lambda i: (0, 0)),
            pl.BlockSpec((D, D), lambda i: (0, 0)),
            pl.BlockSpec((128, D), lambda i: (i, 0)),
        ],
        out_specs=pl.BlockSpec((128, D), lambda i: (i, 0)),
        out_shape=jax.ShapeDtypeStruct((NPAD, D), jnp.float32),
    )(x, stats, g, be, w, coutb)


def _tc_bn_body(x_ref, stats_ref, g_ref, be_ref, out_ref):
    mu = stats_ref[0, :] * (1.0 / N)
    var = stats_ref[1, :] * (1.0 / N) - mu * mu
    scale = g_ref[0] * lax.rsqrt(var + 1e-5)
    out_ref[...] = jnp.tanh(
        (x_ref[...] - mu[None, :]) * scale[None, :] + be_ref[0][None, :])


def _tc_bn(x, stats, g, be):
    grid = NPAD // 128
    return pl.pallas_call(
        _tc_bn_body,
        grid=(grid,),
        in_specs=[
            pl.BlockSpec((128, D), lambda i: (i, 0)),
            pl.BlockSpec((8, D), lambda i: (0, 0)),
            pl.BlockSpec((1, D), lambda i: (0, 0)),
            pl.BlockSpec((1, D), lambda i: (0, 0)),
        ],
        out_specs=pl.BlockSpec((128, D), lambda i: (i, 0)),
        out_shape=jax.ShapeDtypeStruct((NPAD, D), jnp.float32),
    )(x, stats, g, be)


def _tc_score_body(e_ref, r_ref, h_ref, out_ref):
    q = e_ref[...] * r_ref[...]
    logits = lax.dot_general(q, h_ref[...], (((1,), (1,)), ((), ())),
                             preferred_element_type=jnp.float32)
    out_ref[...] = 1.0 / (1.0 + jnp.exp(-logits))


def _tc_score(e1_emb, r_emb, h2):
    grid = NPAD // 128
    return pl.pallas_call(
        _tc_score_body,
        grid=(grid,),
        in_specs=[
            pl.BlockSpec((B, D), lambda j: (0, 0)),
            pl.BlockSpec((B, D), lambda j: (0, 0)),
            pl.BlockSpec((128, D), lambda j: (j, 0)),
        ],
        out_specs=pl.BlockSpec((B, 128), lambda j: (0, j)),
        out_shape=jax.ShapeDtypeStruct((B, NPAD), jnp.float32),
    )(e1_emb, r_emb, h2)


@jax.jit
def kernel(node_feat, edge_index, e1, rel, W0, b0, W1, b1,
           g0, be0, g1, be1, rel_table):
    src = edge_index[0].astype(jnp.int32)
    dst = edge_index[1].astype(jnp.int32)
    pad = E_PAD - E
    src_p = jnp.concatenate([src, jnp.full((pad,), N, jnp.int32)]).reshape(E_ROWS, EW)
    dst_p = jnp.concatenate([dst, jnp.full((pad,), N, jnp.int32)]).reshape(E_ROWS, EW)

    nf_p = jnp.concatenate(
        [node_feat, jnp.zeros((NPAD - N, D), jnp.float32)], axis=0)
    zerosD = jnp.zeros((NPAD, D), jnp.float32)
    onesD = jnp.ones((EW, 128), jnp.float32)
    e1_2d = e1.astype(jnp.int32).reshape(NW, BPT)
    rel_2d = rel.astype(jnp.int32).reshape(NW, BPT)
    b0_2d = b0.reshape(1, D)
    b1_2d = b1.reshape(1, D)
    g0_2d = g0.reshape(1, D)
    g1_2d = g1.reshape(1, D)
    be0_2d = be0.reshape(1, D)
    be1_2d = be1.reshape(1, D)

    degp_out, degp_in = _sc_degrees(src_p, dst_p, onesD, zerosD)
    coutb, cinb = _tc_deg_prep(degp_out, degp_in)

    m0 = _tc_prescale(nf_p, W0, coutb)
    aggp0 = _sc_edge_agg(m0, src_p, dst_p, zerosD)
    x0, stats0 = _tc_postscale(aggp0, cinb, b0_2d)
    m1 = _tc_bn_matmul(x0, stats0, g0_2d, be0_2d, W1, coutb)
    aggp1 = _sc_edge_agg(m1, src_p, dst_p, zerosD)
    x1, stats1 = _tc_postscale(aggp1, cinb, b1_2d)
    h2 = _tc_bn(x1, stats1, g1_2d, be1_2d)

    e1_emb, r_emb = _sc_batch_gather(h2, rel_table, e1_2d, rel_2d)
    pred = _tc_score(e1_emb, r_emb, h2)
    return pred[:, :N]
